# baseline (speedup 1.0000x reference)
import functools

import jax
import jax.numpy as jnp
from jax import lax
from jax.experimental import pallas as pl
from jax.experimental.pallas import tpu as pltpu
from jax.experimental.pallas import tpu_sc as plsc

BATCH = 1024
EMBED = 32
HID1 = 64
HID2 = 32
VOCAB = 100000
TILE = 2048
NB = (VOCAB + TILE - 1) // TILE

_NC = 2
_NS = 16
_NW = _NC * _NS
_BPW = BATCH // _NW


def _sc_gather_body(utab, uid, ptab, pid, uout, pout,
                    uidx_v, urows_v, pidx_v, prows_v, sem):
    wid = lax.axis_index("s") * _NC + lax.axis_index("c")
    base = wid * _BPW
    pltpu.sync_copy(uid.at[pl.ds(base, _BPW)], uidx_v)
    pltpu.sync_copy(pid.at[pl.ds(base, _BPW)], pidx_v)
    pltpu.async_copy(utab.at[uidx_v], urows_v, sem).wait()
    pltpu.async_copy(ptab.at[pidx_v], prows_v, sem).wait()
    pltpu.sync_copy(urows_v, uout.at[pl.ds(base, _BPW)])
    pltpu.sync_copy(prows_v, pout.at[pl.ds(base, _BPW)])


def _stats_body(u, p, w1, b1, w2, b2, w3, b3, h_ref, m_ref, s_ref):
    j = pl.program_id(0)

    @pl.when(j == 0)
    def _init():
        a = u[:] @ w1[0:EMBED, :] + p[:] @ w1[EMBED:2 * EMBED, :] + b1[:]
        a = jnp.maximum(a, 0.0)
        h_ref[:] = jnp.maximum(a @ w2[:] + b2[:], 0.0)
        m_ref[:] = jnp.full((BATCH, 1), -jnp.inf, jnp.float32)
        s_ref[:] = jnp.zeros((BATCH, 1), jnp.float32)

    logits = h_ref[:] @ w3[:] + b3[:]
    col = j * TILE + lax.broadcasted_iota(jnp.int32, (BATCH, TILE), 1)
    logits = jnp.where(col < VOCAB, logits, -jnp.inf)
    m_old = m_ref[:]
    m_new = jnp.maximum(m_old, jnp.max(logits, axis=1, keepdims=True))
    s_ref[:] = (s_ref[:] * jnp.exp(m_old - m_new)
                + jnp.sum(jnp.exp(logits - m_new), axis=1, keepdims=True))
    m_ref[:] = m_new

    @pl.when(j == NB - 1)
    def _fin():
        s_ref[:] = 1.0 / s_ref[:]


def _write_body(h, m, s, w3, b3, o_ref):
    logits = h[:] @ w3[:] + b3[:]
    o_ref[:] = jnp.exp(logits - m[:]) * s[:]


def _const_spec(shape):
    return pl.BlockSpec(shape, lambda j: (0,) * len(shape))


def kernel(user_id, podcast_id, user_table, podcast_table, W1, b1, W2, b2, W3, b3):
    mesh = plsc.VectorSubcoreMesh(core_axis_name="c", subcore_axis_name="s")
    gather = pl.kernel(
        _sc_gather_body,
        out_type=[
            jax.ShapeDtypeStruct((BATCH, EMBED), jnp.float32),
            jax.ShapeDtypeStruct((BATCH, EMBED), jnp.float32),
        ],
        mesh=mesh,
        scratch_types=[
            pltpu.VMEM((_BPW,), jnp.int32),
            pltpu.VMEM((_BPW, EMBED), jnp.float32),
            pltpu.VMEM((_BPW,), jnp.int32),
            pltpu.VMEM((_BPW, EMBED), jnp.float32),
            pltpu.SemaphoreType.DMA,
        ],
        compiler_params=pltpu.CompilerParams(use_tc_tiling_on_sc=False),
    )
    u_emb, p_emb = gather(user_table, user_id, podcast_table, podcast_id)

    b1r = b1.reshape(1, HID1)
    b2r = b2.reshape(1, HID2)
    b3r = b3.reshape(1, VOCAB)

    h, m, s = pl.pallas_call(
        _stats_body,
        grid=(NB,),
        in_specs=[
            _const_spec((BATCH, EMBED)),
            _const_spec((BATCH, EMBED)),
            _const_spec((2 * EMBED, HID1)),
            _const_spec((1, HID1)),
            _const_spec((HID1, HID2)),
            _const_spec((1, HID2)),
            pl.BlockSpec((HID2, TILE), lambda j: (0, j)),
            pl.BlockSpec((1, TILE), lambda j: (0, j)),
        ],
        out_specs=[
            _const_spec((BATCH, HID2)),
            _const_spec((BATCH, 1)),
            _const_spec((BATCH, 1)),
        ],
        out_shape=[
            jax.ShapeDtypeStruct((BATCH, HID2), jnp.float32),
            jax.ShapeDtypeStruct((BATCH, 1), jnp.float32),
            jax.ShapeDtypeStruct((BATCH, 1), jnp.float32),
        ],
    )(u_emb, p_emb, W1, b1r, W2, b2r, W3, b3r)

    out = pl.pallas_call(
        _write_body,
        grid=(NB,),
        in_specs=[
            _const_spec((BATCH, HID2)),
            _const_spec((BATCH, 1)),
            _const_spec((BATCH, 1)),
            pl.BlockSpec((HID2, TILE), lambda j: (0, j)),
            pl.BlockSpec((1, TILE), lambda j: (0, j)),
        ],
        out_specs=pl.BlockSpec((BATCH, TILE), lambda j: (0, j)),
        out_shape=jax.ShapeDtypeStruct((BATCH, VOCAB), jnp.float32),
    )(h, m, s, W3, b3r)
    return out


# speedup vs baseline: 1.2243x; 1.2243x over previous
import functools

import jax
import jax.numpy as jnp
from jax import lax
from jax.experimental import pallas as pl
from jax.experimental.pallas import tpu as pltpu
from jax.experimental.pallas import tpu_sc as plsc

BATCH = 1024
EMBED = 32
HID1 = 64
HID2 = 32
VOCAB = 100000
TILE = 2048
NB = (VOCAB + TILE - 1) // TILE

_NC = 2
_NS = 16
_NW = _NC * _NS
_BPW = BATCH // _NW


def _sc_gather_body(utab, uid, ptab, pid, uout, pout,
                    uidx_v, urows_v, pidx_v, prows_v, usem, psem):
    wid = lax.axis_index("s") * _NC + lax.axis_index("c")
    base = wid * _BPW
    pltpu.sync_copy(uid.at[pl.ds(base, _BPW)], uidx_v)
    pltpu.sync_copy(pid.at[pl.ds(base, _BPW)], pidx_v)
    lanes = jnp.arange(16, dtype=jnp.int32)
    copies = []
    for c in range(_BPW // 16):
        uvec = uidx_v[pl.ds(16 * c, 16)]
        pvec = pidx_v[pl.ds(16 * c, 16)]
        for k in range(16):
            su = jnp.sum(jnp.where(lanes == k, uvec, 0))
            sp = jnp.sum(jnp.where(lanes == k, pvec, 0))
            b = 16 * c + k
            copies.append(pltpu.async_copy(utab.at[su], urows_v.at[b], usem))
            copies.append(pltpu.async_copy(ptab.at[sp], prows_v.at[b], psem))
    for cp in copies:
        cp.wait()
    pltpu.sync_copy(urows_v, uout.at[pl.ds(base, _BPW)])
    pltpu.sync_copy(prows_v, pout.at[pl.ds(base, _BPW)])


def _stats_body(u, p, w1, b1, w2, b2, w3, b3, h_ref, m_ref, s_ref):
    j = pl.program_id(0)

    @pl.when(j == 0)
    def _init():
        a = u[:] @ w1[0:EMBED, :] + p[:] @ w1[EMBED:2 * EMBED, :] + b1[:]
        a = jnp.maximum(a, 0.0)
        h_ref[:] = jnp.maximum(a @ w2[:] + b2[:], 0.0)
        m_ref[:] = jnp.full((BATCH, 1), -jnp.inf, jnp.float32)
        s_ref[:] = jnp.zeros((BATCH, 1), jnp.float32)

    logits = h_ref[:] @ w3[:] + b3[:]
    col = j * TILE + lax.broadcasted_iota(jnp.int32, (BATCH, TILE), 1)
    logits = jnp.where(col < VOCAB, logits, -jnp.inf)
    m_old = m_ref[:]
    m_new = jnp.maximum(m_old, jnp.max(logits, axis=1, keepdims=True))
    s_ref[:] = (s_ref[:] * jnp.exp(m_old - m_new)
                + jnp.sum(jnp.exp(logits - m_new), axis=1, keepdims=True))
    m_ref[:] = m_new

    @pl.when(j == NB - 1)
    def _fin():
        s_ref[:] = 1.0 / s_ref[:]


def _write_body(h, m, s, w3, b3, o_ref):
    logits = h[:] @ w3[:] + b3[:]
    o_ref[:] = jnp.exp(logits - m[:]) * s[:]


def _const_spec(shape):
    return pl.BlockSpec(shape, lambda j: (0,) * len(shape))


def kernel(user_id, podcast_id, user_table, podcast_table, W1, b1, W2, b2, W3, b3):
    mesh = plsc.VectorSubcoreMesh(core_axis_name="c", subcore_axis_name="s")
    gather = pl.kernel(
        _sc_gather_body,
        out_type=[
            jax.ShapeDtypeStruct((BATCH, EMBED), jnp.float32),
            jax.ShapeDtypeStruct((BATCH, EMBED), jnp.float32),
        ],
        mesh=mesh,
        scratch_types=[
            pltpu.VMEM((_BPW,), jnp.int32),
            pltpu.VMEM((_BPW, EMBED), jnp.float32),
            pltpu.VMEM((_BPW,), jnp.int32),
            pltpu.VMEM((_BPW, EMBED), jnp.float32),
            pltpu.SemaphoreType.DMA,
            pltpu.SemaphoreType.DMA,
        ],
        compiler_params=pltpu.CompilerParams(needs_layout_passes=False),
    )
    u_emb, p_emb = gather(user_table, user_id, podcast_table, podcast_id)

    b1r = b1.reshape(1, HID1)
    b2r = b2.reshape(1, HID2)
    b3r = b3.reshape(1, VOCAB)

    h, m, s = pl.pallas_call(
        _stats_body,
        grid=(NB,),
        in_specs=[
            _const_spec((BATCH, EMBED)),
            _const_spec((BATCH, EMBED)),
            _const_spec((2 * EMBED, HID1)),
            _const_spec((1, HID1)),
            _const_spec((HID1, HID2)),
            _const_spec((1, HID2)),
            pl.BlockSpec((HID2, TILE), lambda j: (0, j)),
            pl.BlockSpec((1, TILE), lambda j: (0, j)),
        ],
        out_specs=[
            _const_spec((BATCH, HID2)),
            _const_spec((BATCH, 1)),
            _const_spec((BATCH, 1)),
        ],
        out_shape=[
            jax.ShapeDtypeStruct((BATCH, HID2), jnp.float32),
            jax.ShapeDtypeStruct((BATCH, 1), jnp.float32),
            jax.ShapeDtypeStruct((BATCH, 1), jnp.float32),
        ],
    )(u_emb, p_emb, W1, b1r, W2, b2r, W3, b3r)

    out = pl.pallas_call(
        _write_body,
        grid=(NB,),
        in_specs=[
            _const_spec((BATCH, HID2)),
            _const_spec((BATCH, 1)),
            _const_spec((BATCH, 1)),
            pl.BlockSpec((HID2, TILE), lambda j: (0, j)),
            pl.BlockSpec((1, TILE), lambda j: (0, j)),
        ],
        out_specs=pl.BlockSpec((BATCH, TILE), lambda j: (0, j)),
        out_shape=jax.ShapeDtypeStruct((BATCH, VOCAB), jnp.float32),
    )(h, m, s, W3, b3r)
    return out
